# Initial kernel scaffold; baseline (speedup 1.0000x reference)
import jax
import jax.numpy as jnp
from jax.experimental import pallas as pl


def kernel(x, adj, W1, b1, W2, b2, Wt, bt):
    raise NotImplementedError("write your pallas kernel here")


# speedup vs baseline: 1.0018x; 1.0018x over previous
import jax
import jax.numpy as jnp
from jax import lax
from jax.experimental import pallas as pl


def _gcn_body(x_ref, adj_ref, w1t_ref, b1_ref, w2t_ref, b2_ref, h_ref):
    xb = x_ref[0]
    a = adj_ref[0]
    h = jnp.dot(xb, w1t_ref[...], preferred_element_type=jnp.float32)
    h = h + b1_ref[...]
    h = jnp.maximum(jnp.dot(a, h, preferred_element_type=jnp.float32), 0.0)
    h = jnp.dot(h, w2t_ref[...], preferred_element_type=jnp.float32)
    h = h + b2_ref[...]
    h_ref[0] = jnp.maximum(
        jnp.dot(a, h, preferred_element_type=jnp.float32), 0.0)


def _tok_body(flat_ref, wt_ref, bt_ref, out_ref):
    out = lax.dot_general(
        flat_ref[...], wt_ref[...],
        dimension_numbers=(((1,), (1,)), ((), ())),
        preferred_element_type=jnp.float32)
    out_ref[...] = out + bt_ref[...]


def kernel(x, adj, W1, b1, W2, b2, Wt, bt):
    B, N, F_IN = x.shape
    F_OUT = W1.shape[0]
    w1t = W1.T
    w2t = W2.T
    b1r = b1.reshape(1, F_OUT)
    b2r = b2.reshape(1, F_OUT)
    btr = bt.reshape(1, F_OUT)

    const = lambda shape: pl.BlockSpec(shape, lambda b: tuple(0 for _ in shape))
    h = pl.pallas_call(
        _gcn_body,
        grid=(B,),
        in_specs=[
            pl.BlockSpec((1, N, F_IN), lambda b: (b, 0, 0)),
            pl.BlockSpec((1, N, N), lambda b: (b, 0, 0)),
            const((F_IN, F_OUT)),
            const((1, F_OUT)),
            const((F_OUT, F_OUT)),
            const((1, F_OUT)),
        ],
        out_specs=pl.BlockSpec((1, N, F_OUT), lambda b: (b, 0, 0)),
        out_shape=jax.ShapeDtypeStruct((B, N, F_OUT), jnp.float32),
    )(x, adj, w1t, b1r, w2t, b2r)

    flat = h.reshape(B, N * F_OUT)
    return pl.pallas_call(
        _tok_body,
        in_specs=[
            pl.BlockSpec((B, N * F_OUT), lambda: (0, 0)),
            pl.BlockSpec((F_OUT, N * F_OUT), lambda: (0, 0)),
            pl.BlockSpec((1, F_OUT), lambda: (0, 0)),
        ],
        out_specs=pl.BlockSpec((B, F_OUT), lambda: (0, 0)),
        out_shape=jax.ShapeDtypeStruct((B, F_OUT), jnp.float32),
    )(flat, Wt, btr)
